# baseline (speedup 1.0000x reference)
import functools

import jax
import jax.numpy as jnp
from jax import lax
from jax.experimental import pallas as pl
from jax.experimental.pallas import tpu as pltpu
from jax.experimental.pallas import tpu_sc as plsc

VOCAB = 1000
D = 1000
VP = 1024
B, T = 4096, 20
L = 16
NDT = 4
NKG = B // L

_info = plsc.get_sparse_core_info()
NC, NS = _info.num_cores, _info.num_subcores
NW = NC * NS
N_TILES = D // 8

_mesh = plsc.VectorSubcoreMesh(core_axis_name="c", subcore_axis_name="s")


@functools.partial(
    pl.kernel,
    mesh=_mesh,
    out_type=jax.ShapeDtypeStruct((T, D, B), jnp.float32),
    scratch_types=[
        [pltpu.VMEM((B,), jnp.int32) for _ in range(2)],
        pltpu.VMEM((NDT * 8 * VP,), jnp.float32),
        [pltpu.VMEM((8, B), jnp.float32) for _ in range(2)],
        [pltpu.SemaphoreType.DMA] * 2,
        [pltpu.SemaphoreType.DMA] * 2,
        pltpu.SemaphoreType.DMA,
    ],
    compiler_params=pltpu.CompilerParams(
        use_tc_tiling_on_sc=True, needs_layout_passes=False),
)
def _gather_kernel(idxT_hbm, ttf_hbm, out_hbm, idxs, tts, slabs, isem, wsem,
                   tsem):
    wid = lax.axis_index("s") * NC + lax.axis_index("c")
    dt0 = jnp.minimum(wid * N_TILES // NW, N_TILES - NDT)

    def i_load(t, sl):
        return pltpu.make_async_copy(idxT_hbm.at[t], idxs[sl], isem[sl])

    def w_copy(t, dt, sl):
        return pltpu.make_async_copy(
            slabs[sl], out_hbm.at[t, pl.ds((dt0 + dt) * 8, 8)], wsem[sl])

    def build_and_write(t, dt, sl):
        w_copy(0, 0, sl).wait()

        def per_kg(k, carry):
            iv = idxs[sl][pl.ds(L * k, L)]
            xs = [plsc.load_gather(
                tts.at[pl.ds((dt * 8 + i) * VP, VP)], [iv]) for i in range(8)]
            for i in range(8):
                slabs[sl][i, pl.ds(L * k, L)] = xs[i]
            return carry

        lax.fori_loop(0, NKG, per_kg, 0, unroll=4)
        w_copy(t, dt, sl).start()

    pltpu.make_async_copy(
        ttf_hbm.at[pl.ds(dt0 * (8 * VP), NDT * 8 * VP)], tts, tsem).start()

    w_copy(0, 0, 0).start()
    w_copy(0, 1, 1).start()

    i_load(0, 0).start()
    i_load(1, 1).start()
    pltpu.make_async_copy(ttf_hbm.at[pl.ds(0, NDT * 8 * VP)], tts, tsem).wait()

    def t_pair(p, carry):
        for sl in range(2):
            t = 2 * p + sl
            i_load(0, sl).wait()
            for dt in range(NDT):
                build_and_write(t, dt, dt & 1)

            @pl.when(p < T // 2 - 1)
            def _():
                i_load(t + 2, sl).start()
        return carry

    lax.fori_loop(0, T // 2, t_pair, 0)

    w_copy(0, 0, 0).wait()
    w_copy(0, 0, 1).wait()


def kernel(index, table):
    idxT = index.T.astype(jnp.int32)
    ttf = jnp.pad(table.T, ((0, 0), (0, VP - VOCAB))).reshape(-1)
    out_phys = _gather_kernel(idxT, ttf)
    return jnp.transpose(out_phys, (2, 0, 1))
